# baseline (speedup 1.0000x reference)
import functools

import jax
import jax.numpy as jnp
from jax import lax
from jax.experimental import pallas as pl
from jax.experimental.pallas import tpu as pltpu
from jax.experimental.pallas import tpu_sc as plsc

ROWS = 64
N = 8192
K_OUT = 64
NCHUNK = N // 16
INT_MIN = -2147483648


def _sort16_desc(v):
    k, _ = plsc.sort_key_val(v, v, descending=True)
    return k


def _rev(v):
    return lax.rev(v, (0,))


def _sort64_desc(v0, v1, v2, v3):
    a0, a1, a2, a3 = (_sort16_desc(v) for v in (v0, v1, v2, v3))

    def merge2(a, b):
        rb = _rev(b)
        return _sort16_desc(jnp.maximum(a, rb)), _sort16_desc(jnp.minimum(a, rb))

    a0, a1 = merge2(a0, a1)
    b0, b1 = merge2(a2, a3)
    rb0, rb1 = _rev(b1), _rev(b0)
    hi0, hi1 = jnp.maximum(a0, rb0), jnp.maximum(a1, rb1)
    lo0, lo1 = jnp.minimum(a0, rb0), jnp.minimum(a1, rb1)
    u0, u1 = jnp.maximum(hi0, hi1), jnp.minimum(hi0, hi1)
    u2, u3 = jnp.maximum(lo0, lo1), jnp.minimum(lo0, lo1)
    return (_sort16_desc(u0), _sort16_desc(u1),
            _sort16_desc(u2), _sort16_desc(u3))


def _sc_topk_body(x_hbm, out_hbm, xv, keys, outv, sem):
    del sem
    wid = lax.axis_index("s") * 2 + lax.axis_index("c")
    lane = lax.broadcasted_iota(jnp.int32, (16,), 0)

    def do_row(j, _):
        row = wid * 2 + j
        pltpu.sync_copy(x_hbm.at[row], xv)

        def build(i, carry):
            x16 = xv[pl.ds(i * 16, 16)]
            b = lax.bitcast_convert_type(x16, jnp.int32)
            k16 = jnp.where(b < 0, ~b, b | jnp.int32(INT_MIN))
            keys[pl.ds(i * 16, 16)] = k16
            return carry

        lax.fori_loop(0, NCHUNK, build, 0)

        src = jnp.int32(0)
        d1 = jnp.int32(N)
        d0 = jnp.int32(2 * N)
        n_cur = jnp.int32(N)
        r_left = jnp.int32(K_OUT)
        pfx = jnp.int32(0)

        for t in range(31, -1, -1):
            m = jnp.int32(INT_MIN if t == 31 else (1 << t))
            nch = (n_cur + 15) // 16

            def part(i, carry, src=src, m=m, n_cur=n_cur):
                off1, off0 = carry
                k16 = keys[pl.ds(src + i * 16, 16)]
                valid = (i * 16 + lane) < n_cur
                bit = (k16 & m) != 0
                m1 = jnp.logical_and(bit, valid)
                m0 = jnp.logical_and(jnp.logical_not(bit), valid)
                c1 = plsc.cumsum(m1.astype(jnp.int32))
                plsc.store_scatter(keys, [off1 + c1 - 1], k16, mask=m1)
                c0 = plsc.cumsum(m0.astype(jnp.int32))
                plsc.store_scatter(keys, [off0 + c0 - 1], k16, mask=m0)
                off1 = off1 + plsc.all_reduce_population_count(m1)
                off0 = off0 + plsc.all_reduce_population_count(m0)
                return off1, off0

            off1, off0 = lax.fori_loop(
                0, nch, part,
                (jnp.full((16,), d1, jnp.int32), jnp.full((16,), d0, jnp.int32)))
            cnt1 = jnp.max(off1) - d1
            take1 = cnt1 >= r_left
            pfx = jnp.where(take1, pfx | m, pfx)
            r_left = jnp.where(take1, r_left, r_left - cnt1)
            n_new = jnp.where(take1, cnt1, n_cur - cnt1)
            src_new = jnp.where(take1, d1, d0)
            d0_new = jnp.where(take1, d0, d1)
            d1, d0, src, n_cur = src, d0_new, src_new, n_new

        pb = jnp.where(pfx < 0, pfx & jnp.int32(0x7FFFFFFF), ~pfx)
        v64 = lax.bitcast_convert_type(jnp.full((16,), pb, jnp.int32), jnp.float32)

        for q in range(K_OUT // 16):
            outv[pl.ds(q * 16, 16)] = v64

        def compact(i, off):
            x16 = xv[pl.ds(i * 16, 16)]
            msk = x16 > v64
            c = plsc.cumsum(msk.astype(jnp.int32))
            plsc.store_scatter(outv, [off + c - 1], x16, mask=msk)
            return off + plsc.all_reduce_population_count(msk)

        lax.fori_loop(0, NCHUNK, compact, jnp.zeros((16,), jnp.int32))

        s0, s1, s2, s3 = _sort64_desc(
            outv[pl.ds(0, 16)], outv[pl.ds(16, 16)],
            outv[pl.ds(32, 16)], outv[pl.ds(48, 16)])
        outv[pl.ds(0, 16)] = s0
        outv[pl.ds(16, 16)] = s1
        outv[pl.ds(32, 16)] = s2
        outv[pl.ds(48, 16)] = s3
        pltpu.sync_copy(outv, out_hbm.at[row])
        return _

    lax.fori_loop(0, 2, do_row, 0)


@jax.jit
def kernel(inputs):
    mesh = plsc.VectorSubcoreMesh(core_axis_name="c", subcore_axis_name="s")
    f = functools.partial(
        pl.kernel,
        mesh=mesh,
        compiler_params=pltpu.CompilerParams(needs_layout_passes=False),
        out_type=jax.ShapeDtypeStruct((ROWS, K_OUT), jnp.float32),
        scratch_types=[
            pltpu.VMEM((N,), jnp.float32),
            pltpu.VMEM((3 * N,), jnp.int32),
            pltpu.VMEM((K_OUT,), jnp.float32),
            pltpu.SemaphoreType.DMA,
        ],
    )(_sc_topk_body)
    return f(inputs)


# speedup vs baseline: 1.0202x; 1.0202x over previous
import functools

import jax
import jax.numpy as jnp
from jax import lax
from jax.experimental import pallas as pl
from jax.experimental.pallas import tpu as pltpu
from jax.experimental.pallas import tpu_sc as plsc

ROWS = 64
N = 8192
K_OUT = 64
INT_MIN = -2147483648
U = 4


def _sort16_desc(v):
    k, _ = plsc.sort_key_val(v, v, descending=True)
    return k


def _rev(v):
    return lax.rev(v, (0,))


def _sort64_desc(v0, v1, v2, v3):
    a0, a1, a2, a3 = (_sort16_desc(v) for v in (v0, v1, v2, v3))

    def merge2(a, b):
        rb = _rev(b)
        return _sort16_desc(jnp.maximum(a, rb)), _sort16_desc(jnp.minimum(a, rb))

    a0, a1 = merge2(a0, a1)
    b0, b1 = merge2(a2, a3)
    rb0, rb1 = _rev(b1), _rev(b0)
    hi0, hi1 = jnp.maximum(a0, rb0), jnp.maximum(a1, rb1)
    lo0, lo1 = jnp.minimum(a0, rb0), jnp.minimum(a1, rb1)
    u0, u1 = jnp.maximum(hi0, hi1), jnp.minimum(hi0, hi1)
    u2, u3 = jnp.maximum(lo0, lo1), jnp.minimum(lo0, lo1)
    return (_sort16_desc(u0), _sort16_desc(u1),
            _sort16_desc(u2), _sort16_desc(u3))


def _to_key(x16):
    b = lax.bitcast_convert_type(x16, jnp.int32)
    return jnp.where(b < 0, ~b, b | jnp.int32(INT_MIN))


def _sc_topk_body(x_hbm, out_hbm, xv, keys, outv, sem):
    del sem
    wid = lax.axis_index("s") * 2 + lax.axis_index("c")
    lane = lax.broadcasted_iota(jnp.int32, (16,), 0)

    def do_row(j, _):
        row = wid * 2 + j
        pltpu.sync_copy(x_hbm.at[row], xv)

        m31 = jnp.int32(INT_MIN)

        def part31(i, carry):
            off1, off0 = carry
            for u in range(U):
                x16 = xv[pl.ds(i * (16 * U) + u * 16, 16)]
                k16 = _to_key(x16)
                bit = (k16 & m31) != 0
                nbit = jnp.logical_not(bit)
                c1 = plsc.cumsum(bit.astype(jnp.int32))
                plsc.store_scatter(keys, [off1 + c1 - 1], k16, mask=bit)
                c0 = plsc.cumsum(nbit.astype(jnp.int32))
                plsc.store_scatter(keys, [off0 + c0 - 1], k16, mask=nbit)
                off1 = off1 + plsc.all_reduce_population_count(bit)
                off0 = off0 + plsc.all_reduce_population_count(nbit)
            return off1, off0

        off1, off0 = lax.fori_loop(
            0, N // (16 * U), part31,
            (jnp.full((16,), 0, jnp.int32), jnp.full((16,), N, jnp.int32)))
        cnt1 = jnp.max(off1)
        r_left = jnp.int32(K_OUT)
        take1 = cnt1 >= r_left
        pfx = jnp.where(take1, m31, jnp.int32(0))
        r_left = jnp.where(take1, r_left, r_left - cnt1)
        n_cur = jnp.where(take1, cnt1, jnp.int32(N) - cnt1)
        src = jnp.where(take1, jnp.int32(0), jnp.int32(N))
        d1 = jnp.int32(2 * N)
        d0 = jnp.where(take1, jnp.int32(N), jnp.int32(0))

        for t in range(30, -1, -1):
            m = jnp.int32(1 << t)
            nch = (n_cur + (16 * U - 1)) // (16 * U)

            def part(i, carry, src=src, m=m, n_cur=n_cur):
                off1, off0 = carry
                for u in range(U):
                    base = i * (16 * U) + u * 16
                    k16 = keys[pl.ds(src + base, 16)]
                    valid = (base + lane) < n_cur
                    bit = (k16 & m) != 0
                    m1 = jnp.logical_and(bit, valid)
                    m0 = jnp.logical_and(jnp.logical_not(bit), valid)
                    c1 = plsc.cumsum(m1.astype(jnp.int32))
                    plsc.store_scatter(keys, [off1 + c1 - 1], k16, mask=m1)
                    c0 = plsc.cumsum(m0.astype(jnp.int32))
                    plsc.store_scatter(keys, [off0 + c0 - 1], k16, mask=m0)
                    off1 = off1 + plsc.all_reduce_population_count(m1)
                    off0 = off0 + plsc.all_reduce_population_count(m0)
                return off1, off0

            off1, off0 = lax.fori_loop(
                0, nch, part,
                (jnp.full((16,), d1, jnp.int32), jnp.full((16,), d0, jnp.int32)))
            cnt1 = jnp.max(off1) - d1
            take1 = cnt1 >= r_left
            pfx = jnp.where(take1, pfx | m, pfx)
            r_left = jnp.where(take1, r_left, r_left - cnt1)
            n_new = jnp.where(take1, cnt1, n_cur - cnt1)
            src_new = jnp.where(take1, d1, d0)
            d0_new = jnp.where(take1, d0, d1)
            d1, d0, src, n_cur = src, d0_new, src_new, n_new

        pb = jnp.where(pfx < 0, pfx & jnp.int32(0x7FFFFFFF), ~pfx)
        v64 = lax.bitcast_convert_type(jnp.full((16,), pb, jnp.int32), jnp.float32)

        for q in range(K_OUT // 16):
            outv[pl.ds(q * 16, 16)] = v64

        def compact(i, off):
            for u in range(U):
                x16 = xv[pl.ds(i * (16 * U) + u * 16, 16)]
                msk = x16 > v64
                c = plsc.cumsum(msk.astype(jnp.int32))
                plsc.store_scatter(outv, [off + c - 1], x16, mask=msk)
                off = off + plsc.all_reduce_population_count(msk)
            return off

        lax.fori_loop(0, N // (16 * U), compact, jnp.zeros((16,), jnp.int32))

        s0, s1, s2, s3 = _sort64_desc(
            outv[pl.ds(0, 16)], outv[pl.ds(16, 16)],
            outv[pl.ds(32, 16)], outv[pl.ds(48, 16)])
        outv[pl.ds(0, 16)] = s0
        outv[pl.ds(16, 16)] = s1
        outv[pl.ds(32, 16)] = s2
        outv[pl.ds(48, 16)] = s3
        pltpu.sync_copy(outv, out_hbm.at[row])
        return _

    lax.fori_loop(0, 2, do_row, 0)


@jax.jit
def kernel(inputs):
    mesh = plsc.VectorSubcoreMesh(core_axis_name="c", subcore_axis_name="s")
    f = functools.partial(
        pl.kernel,
        mesh=mesh,
        compiler_params=pltpu.CompilerParams(needs_layout_passes=False),
        out_type=jax.ShapeDtypeStruct((ROWS, K_OUT), jnp.float32),
        scratch_types=[
            pltpu.VMEM((N,), jnp.float32),
            pltpu.VMEM((3 * N + 16 * U,), jnp.int32),
            pltpu.VMEM((K_OUT,), jnp.float32),
            pltpu.SemaphoreType.DMA,
        ],
    )(_sc_topk_body)
    return f(inputs)
